# baseline (speedup 1.0000x reference)
import functools

import jax
import jax.numpy as jnp
from jax import lax
from jax.experimental import pallas as pl
from jax.experimental.pallas import tpu as pltpu
from jax.experimental.pallas import tpu_sc as plsc

N = 10000
D = 128
NC = 2
NS = 16
NW = NC * NS
CH = 128
NPAD = 10112
RPT = NPAD // NS
NBUF = 3
_ROWCHUNKS = [(off, min(CH, RPT - off)) for off in range(0, RPT, CH)]

_mesh = plsc.VectorSubcoreMesh(core_axis_name="c", subcore_axis_name="s")


def _zero_acc(zeros_hbm, zb, acc, s):
    pltpu.sync_copy(zeros_hbm, zb)
    for off, sz in _ROWCHUNKS:
        pltpu.sync_copy(zb.at[pl.ds(0, sz), :],
                        acc.at[pl.ds(s * RPT + off, sz), :])


def _copy_out(out_hbm, zb, acc, c, s):
    for off, sz in _ROWCHUNKS:
        r0 = s * RPT + off
        pltpu.sync_copy(acc.at[pl.ds(r0, sz), :], zb.at[pl.ds(0, sz), :])
        pltpu.sync_copy(zb.at[pl.ds(0, sz), :], out_hbm.at[c, pl.ds(r0, sz), :])


def _deg_body(edg_hbm, ones_hbm, zeros_hbm, out_hbm,
              i0, i1, i2, ones_v, dacc, s0, s1, s2):
    c = lax.axis_index("c")
    s = lax.axis_index("s")
    wid = s * NC + c
    cpw = edg_hbm.shape[0] // NW
    idx = [i0, i1, i2]
    sems = [s0, s1, s2]
    _zero_acc(zeros_hbm, ones_v, dacc, s)
    pltpu.sync_copy(ones_hbm, ones_v)
    plsc.subcore_barrier()

    base = wid * cpw

    def scat_start(b):
        pltpu.async_copy(ones_v, dacc.at[idx[b].at[1]], sems[b], add=True)

    def scat_wait(b):
        pltpu.make_async_copy(ones_v, dacc.at[idx[b].at[1]], sems[b]).wait()

    def step(t, carry):
        for b in range(NBUF):
            g = t * NBUF + b

            @pl.when(t > 0)
            def _():
                scat_wait(b)

            pltpu.sync_copy(edg_hbm.at[base + g], idx[b])
            scat_start(b)
        return carry

    lax.fori_loop(0, cpw // NBUF, step, 0)
    for b in range(NBUF):
        scat_wait(b)
    plsc.subcore_barrier()
    _copy_out(out_hbm, ones_v, dacc, c, s)


_deg_call = functools.partial(
    pl.kernel,
    out_type=jax.ShapeDtypeStruct((NC, NPAD, D), jnp.float32),
    mesh=_mesh,
    scratch_types=[
        pltpu.VMEM((2, CH), jnp.int32),
        pltpu.VMEM((2, CH), jnp.int32),
        pltpu.VMEM((2, CH), jnp.int32),
        pltpu.VMEM((CH, D), jnp.float32),
        pltpu.VMEM_SHARED((NPAD, D), jnp.float32),
        pltpu.SemaphoreType.DMA,
        pltpu.SemaphoreType.DMA,
        pltpu.SemaphoreType.DMA,
    ],
)(_deg_body)


def _agg_body(u_hbm, edg_hbm, zeros_hbm, out_hbm,
              i0, i1, i2, r0_, r1_, r2_, acc,
              g0, g1, g2, s0, s1, s2):
    c = lax.axis_index("c")
    s = lax.axis_index("s")
    wid = s * NC + c
    cpw = edg_hbm.shape[0] // NW
    idx = [i0, i1, i2]
    rows = [r0_, r1_, r2_]
    semg = [g0, g1, g2]
    sems = [s0, s1, s2]
    _zero_acc(zeros_hbm, rows[0], acc, s)
    plsc.subcore_barrier()

    base = wid * cpw

    def gath_start(b):
        pltpu.async_copy(u_hbm.at[idx[b].at[0]], rows[b], semg[b])

    def gath_wait(b):
        pltpu.make_async_copy(u_hbm.at[idx[b].at[0]], rows[b], semg[b]).wait()

    def scat_start(b):
        pltpu.async_copy(rows[b], acc.at[idx[b].at[1]], sems[b], add=True)

    def scat_wait(b):
        pltpu.make_async_copy(rows[b], acc.at[idx[b].at[1]], sems[b]).wait()

    for g in range(2):
        pltpu.sync_copy(edg_hbm.at[base + g], idx[g])
        gath_start(g)

    def step(t, carry):
        for b in range(NBUF):
            g = t * NBUF + b
            ga = g + 2
            bf = (b + 2) % NBUF

            @pl.when((ga < cpw) & (g >= 1))
            def _():
                scat_wait(bf)

            @pl.when(ga < cpw)
            def _():
                pltpu.sync_copy(edg_hbm.at[base + ga], idx[bf])
                gath_start(bf)

            gath_wait(b)
            scat_start(b)
        return carry

    lax.fori_loop(0, cpw // NBUF, step, 0)
    for b in range(NBUF):
        scat_wait(b)
    plsc.subcore_barrier()
    _copy_out(out_hbm, rows[0], acc, c, s)


_agg_call = functools.partial(
    pl.kernel,
    out_type=jax.ShapeDtypeStruct((NC, NPAD, D), jnp.float32),
    mesh=_mesh,
    scratch_types=[
        pltpu.VMEM((2, CH), jnp.int32),
        pltpu.VMEM((2, CH), jnp.int32),
        pltpu.VMEM((2, CH), jnp.int32),
        pltpu.VMEM((CH, D), jnp.float32),
        pltpu.VMEM((CH, D), jnp.float32),
        pltpu.VMEM((CH, D), jnp.float32),
        pltpu.VMEM_SHARED((NPAD, D), jnp.float32),
        pltpu.SemaphoreType.DMA,
        pltpu.SemaphoreType.DMA,
        pltpu.SemaphoreType.DMA,
        pltpu.SemaphoreType.DMA,
        pltpu.SemaphoreType.DMA,
        pltpu.SemaphoreType.DMA,
    ],
)(_agg_body)


def _dinv_from(degp_ref):
    deg = degp_ref[0][:, 0:1] + degp_ref[1][:, 0:1] + 1.0
    return lax.rsqrt(deg)


def _tc_first(xp_ref, w_ref, degp_ref, u_ref):
    dinv = _dinv_from(degp_ref)
    h = jnp.dot(xp_ref[...], w_ref[...], preferred_element_type=jnp.float32)
    u_ref[...] = h * dinv


def _pairnorm(h, mask):
    h = h * mask
    cm = jnp.sum(h, axis=0, keepdims=True) * (1.0 / N)
    xc = (h - cm) * mask
    r2 = jnp.sum(xc * xc) * (1.0 / N)
    return xc * lax.rsqrt(1e-6 + r2)


def _tc_mid(p_ref, u_ref, degp_ref, b_ref, w_ref, mask_ref, out_ref):
    dinv = _dinv_from(degp_ref)
    h = (p_ref[0] + p_ref[1] + u_ref[...]) * dinv + b_ref[...]
    y = jnp.maximum(_pairnorm(h, mask_ref[...]), 0.0)
    out_ref[...] = jnp.dot(
        y, w_ref[...], preferred_element_type=jnp.float32) * dinv


def _tc_last(p_ref, u_ref, degp_ref, b_ref, mask_ref, out_ref):
    dinv = _dinv_from(degp_ref)
    h = (p_ref[0] + p_ref[1] + u_ref[...]) * dinv + b_ref[...]
    out_ref[...] = _pairnorm(h, mask_ref[...])


_of32 = jax.ShapeDtypeStruct((NPAD, D), jnp.float32)
_tc_first_call = pl.pallas_call(_tc_first, out_shape=_of32)
_tc_mid_call = pl.pallas_call(_tc_mid, out_shape=_of32)
_tc_last_call = pl.pallas_call(_tc_last, out_shape=_of32)


@jax.jit
def _run(x, edge_index, W0, b0, W1, b1, W2, b2):
    E = edge_index.shape[1]
    cpw = -(-E // (NW * CH))
    cpw = -(-cpw // NBUF) * NBUF
    epad = NW * cpw * CH
    fill = jnp.full((2, epad - E), N, dtype=jnp.int32)
    edg = jnp.concatenate([edge_index.astype(jnp.int32), fill], axis=1)
    edg = jnp.transpose(edg.reshape(2, NW * cpw, CH), (1, 0, 2))

    xp = jnp.pad(x, ((0, NPAD - N), (0, 0)))
    W2p = jnp.pad(W2, ((0, 0), (0, D - W2.shape[1])))
    b0p = jnp.reshape(b0, (1, D))
    b1p = jnp.reshape(b1, (1, D))
    b2p = jnp.reshape(jnp.pad(b2, (0, D - b2.shape[0])), (1, D))
    mask = (jnp.arange(NPAD) < N).astype(jnp.float32)[:, None]
    zeros = jnp.zeros((CH, D), jnp.float32)
    ones = jnp.ones((CH, D), jnp.float32)

    degp = _deg_call(edg, ones, zeros)

    u = _tc_first_call(xp, W0, degp)
    p = _agg_call(u, edg, zeros)
    u = _tc_mid_call(p, u, degp, b0p, W1, mask)
    p = _agg_call(u, edg, zeros)
    u = _tc_mid_call(p, u, degp, b1p, W2p, mask)
    p = _agg_call(u, edg, zeros)
    out = _tc_last_call(p, u, degp, b2p, mask)
    return out[:N, :121]


def kernel(x, edge_index, W0, b0, W1, b1, W2, b2):
    return _run(x, edge_index, W0, b0, W1, b1, W2, b2)


# speedup vs baseline: 1.0038x; 1.0038x over previous
import functools

import jax
import jax.numpy as jnp
from jax import lax
from jax.experimental import pallas as pl
from jax.experimental.pallas import tpu as pltpu
from jax.experimental.pallas import tpu_sc as plsc

N = 10000
D = 128
NC = 2
NS = 16
NW = NC * NS
CH = 128
NPAD = 10112
RPT = NPAD // NS
NBUF = 3
SPLIT0 = 0.76
_ROWCHUNKS = [(off, min(CH, RPT - off)) for off in range(0, RPT, CH)]

_mesh = plsc.VectorSubcoreMesh(core_axis_name="c", subcore_axis_name="s")


def _zero_acc(zeros_hbm, zb, acc, s):
    pltpu.sync_copy(zeros_hbm, zb)
    for off, sz in _ROWCHUNKS:
        pltpu.sync_copy(zb.at[pl.ds(0, sz), :],
                        acc.at[pl.ds(s * RPT + off, sz), :])


def _copy_out(out_hbm, zb, acc, c, s):
    for off, sz in _ROWCHUNKS:
        r0 = s * RPT + off
        pltpu.sync_copy(acc.at[pl.ds(r0, sz), :], zb.at[pl.ds(0, sz), :])
        pltpu.sync_copy(zb.at[pl.ds(0, sz), :], out_hbm.at[c, pl.ds(r0, sz), :])


def _deg_body(edg_hbm, ones_hbm, zeros_hbm, out_hbm,
              i0, i1, i2, ones_v, dacc, s0, s1, s2):
    c = lax.axis_index("c")
    s = lax.axis_index("s")
    wid = s * NC + c
    cpw = edg_hbm.shape[0] // NW
    idx = [i0, i1, i2]
    sems = [s0, s1, s2]
    _zero_acc(zeros_hbm, ones_v, dacc, s)
    pltpu.sync_copy(ones_hbm, ones_v)
    plsc.subcore_barrier()

    base = wid * cpw

    def scat_start(b):
        pltpu.async_copy(ones_v, dacc.at[idx[b].at[1]], sems[b], add=True)

    def scat_wait(b):
        pltpu.make_async_copy(ones_v, dacc.at[idx[b].at[1]], sems[b]).wait()

    def step(t, carry):
        for b in range(NBUF):
            g = t * NBUF + b

            @pl.when(t > 0)
            def _():
                scat_wait(b)

            pltpu.sync_copy(edg_hbm.at[base + g], idx[b])
            scat_start(b)
        return carry

    lax.fori_loop(0, cpw // NBUF, step, 0)
    for b in range(NBUF):
        scat_wait(b)
    plsc.subcore_barrier()
    _copy_out(out_hbm, ones_v, dacc, c, s)


_deg_call = functools.partial(
    pl.kernel,
    out_type=jax.ShapeDtypeStruct((NC, NPAD, D), jnp.float32),
    mesh=_mesh,
    scratch_types=[
        pltpu.VMEM((2, CH), jnp.int32),
        pltpu.VMEM((2, CH), jnp.int32),
        pltpu.VMEM((2, CH), jnp.int32),
        pltpu.VMEM((CH, D), jnp.float32),
        pltpu.VMEM_SHARED((NPAD, D), jnp.float32),
        pltpu.SemaphoreType.DMA,
        pltpu.SemaphoreType.DMA,
        pltpu.SemaphoreType.DMA,
    ],
)(_deg_body)


def _make_agg_body(A, B):
    def _agg_body(u_hbm, edg_hbm, zeros_hbm, out_hbm,
                  i0, i1, i2, r0_, r1_, r2_, acc,
                  g0, g1, g2, s0, s1, s2):
        c = lax.axis_index("c")
        s = lax.axis_index("s")
        idx = [i0, i1, i2]
        rows = [r0_, r1_, r2_]
        semg = [g0, g1, g2]
        sems = [s0, s1, s2]
        _zero_acc(zeros_hbm, rows[0], acc, s)
        plsc.subcore_barrier()

        cnt = jnp.where(c == 0, A, B)
        base = jnp.where(c == 0, s * A, NS * A + s * B)

        def gath_start(b):
            pltpu.async_copy(u_hbm.at[idx[b].at[0]], rows[b], semg[b])

        def gath_wait(b):
            pltpu.make_async_copy(
                u_hbm.at[idx[b].at[0]], rows[b], semg[b]).wait()

        def scat_start(b):
            pltpu.async_copy(rows[b], acc.at[idx[b].at[1]], sems[b], add=True)

        def scat_wait(b):
            pltpu.make_async_copy(
                rows[b], acc.at[idx[b].at[1]], sems[b]).wait()

        @pl.when(cnt > 0)
        def _():
            for g in range(2):
                pltpu.sync_copy(edg_hbm.at[base + g], idx[g])
                gath_start(g)

        def step(t, carry):
            for b in range(NBUF):
                g = t * NBUF + b
                ga = g + 2
                bf = (b + 2) % NBUF

                @pl.when((ga < cnt) & (g >= 1))
                def _():
                    scat_wait(bf)

                @pl.when(ga < cnt)
                def _():
                    pltpu.sync_copy(edg_hbm.at[base + ga], idx[bf])
                    gath_start(bf)

                gath_wait(b)
                scat_start(b)
            return carry

        lax.fori_loop(0, cnt // NBUF, step, 0, unroll=False)

        @pl.when(cnt > 0)
        def _():
            for b in range(NBUF):
                scat_wait(b)

        plsc.subcore_barrier()
        _copy_out(out_hbm, rows[0], acc, c, s)

    return _agg_body


@functools.lru_cache(maxsize=None)
def _agg_call_for(A, B):
    return pl.kernel(
        _make_agg_body(A, B),
        out_type=jax.ShapeDtypeStruct((NC, NPAD, D), jnp.float32),
        mesh=_mesh,
        scratch_types=[
            pltpu.VMEM((2, CH), jnp.int32),
            pltpu.VMEM((2, CH), jnp.int32),
            pltpu.VMEM((2, CH), jnp.int32),
            pltpu.VMEM((CH, D), jnp.float32),
            pltpu.VMEM((CH, D), jnp.float32),
            pltpu.VMEM((CH, D), jnp.float32),
            pltpu.VMEM_SHARED((NPAD, D), jnp.float32),
            pltpu.SemaphoreType.DMA,
            pltpu.SemaphoreType.DMA,
            pltpu.SemaphoreType.DMA,
            pltpu.SemaphoreType.DMA,
            pltpu.SemaphoreType.DMA,
            pltpu.SemaphoreType.DMA,
        ],
    )


def _dinv_from(degp_ref):
    deg = degp_ref[0][:, 0:1] + degp_ref[1][:, 0:1] + 1.0
    return lax.rsqrt(deg)


def _tc_first(xp_ref, w_ref, degp_ref, u_ref):
    dinv = _dinv_from(degp_ref)
    h = jnp.dot(xp_ref[...], w_ref[...], preferred_element_type=jnp.float32)
    u_ref[...] = h * dinv


def _pairnorm(h, mask):
    h = h * mask
    cm = jnp.sum(h, axis=0, keepdims=True) * (1.0 / N)
    xc = (h - cm) * mask
    r2 = jnp.sum(xc * xc) * (1.0 / N)
    return xc * lax.rsqrt(1e-6 + r2)


def _tc_mid(p_ref, u_ref, degp_ref, b_ref, w_ref, mask_ref, out_ref):
    dinv = _dinv_from(degp_ref)
    h = (p_ref[0] + p_ref[1] + u_ref[...]) * dinv + b_ref[...]
    y = jnp.maximum(_pairnorm(h, mask_ref[...]), 0.0)
    out_ref[...] = jnp.dot(
        y, w_ref[...], preferred_element_type=jnp.float32) * dinv


def _tc_last(p_ref, u_ref, degp_ref, b_ref, mask_ref, out_ref):
    dinv = _dinv_from(degp_ref)
    h = (p_ref[0] + p_ref[1] + u_ref[...]) * dinv + b_ref[...]
    out_ref[...] = _pairnorm(h, mask_ref[...])


_of32 = jax.ShapeDtypeStruct((NPAD, D), jnp.float32)
_tc_first_call = pl.pallas_call(_tc_first, out_shape=_of32)
_tc_mid_call = pl.pallas_call(_tc_mid, out_shape=_of32)
_tc_last_call = pl.pallas_call(_tc_last, out_shape=_of32)


@jax.jit
def _run(x, edge_index, W0, b0, W1, b1, W2, b2):
    E = edge_index.shape[1]
    cpw = -(-E // (NW * CH))
    cpw = -(-cpw // NBUF) * NBUF
    epad = NW * cpw * CH
    fill = jnp.full((2, epad - E), N, dtype=jnp.int32)
    edg = jnp.concatenate([edge_index.astype(jnp.int32), fill], axis=1)
    edg = jnp.transpose(edg.reshape(2, NW * cpw, CH), (1, 0, 2))

    xp = jnp.pad(x, ((0, NPAD - N), (0, 0)))
    W2p = jnp.pad(W2, ((0, 0), (0, D - W2.shape[1])))
    b0p = jnp.reshape(b0, (1, D))
    b1p = jnp.reshape(b1, (1, D))
    b2p = jnp.reshape(jnp.pad(b2, (0, D - b2.shape[0])), (1, D))
    mask = (jnp.arange(NPAD) < N).astype(jnp.float32)[:, None]
    zeros = jnp.zeros((CH, D), jnp.float32)
    ones = jnp.ones((CH, D), jnp.float32)

    per_pair = 2 * cpw
    A = int(round(SPLIT0 * per_pair / NBUF)) * NBUF
    A = min(max(A, 0), per_pair)
    B = per_pair - A
    _agg = _agg_call_for(A, B)

    degp = _deg_call(edg, ones, zeros)

    u = _tc_first_call(xp, W0, degp)
    p = _agg(u, edg, zeros)
    u = _tc_mid_call(p, u, degp, b0p, W1, mask)
    p = _agg(u, edg, zeros)
    u = _tc_mid_call(p, u, degp, b1p, W2p, mask)
    p = _agg(u, edg, zeros)
    out = _tc_last_call(p, u, degp, b2p, mask)
    return out[:N, :121]


def kernel(x, edge_index, W0, b0, W1, b1, W2, b2):
    return _run(x, edge_index, W0, b0, W1, b1, W2, b2)


# speedup vs baseline: 4.2067x; 4.1909x over previous
import functools

import jax
import jax.numpy as jnp
from jax import lax
from jax.experimental import pallas as pl
from jax.experimental.pallas import tpu as pltpu
from jax.experimental.pallas import tpu_sc as plsc

N = 10000
D = 128
NC = 2
NS = 16
NW = NC * NS
CH = 128
NPAD = 10112
RPT = NPAD // NS
NBUF = 3
SPLIT0 = 0.5
_ROWCHUNKS = [(off, min(CH, RPT - off)) for off in range(0, RPT, CH)]

_mesh = plsc.VectorSubcoreMesh(core_axis_name="c", subcore_axis_name="s")


def _zero_acc(zeros_hbm, zb, acc, s):
    pltpu.sync_copy(zeros_hbm, zb)
    for off, sz in _ROWCHUNKS:
        pltpu.sync_copy(zb.at[pl.ds(0, sz), :],
                        acc.at[pl.ds(s * RPT + off, sz), :])


def _copy_out(out_hbm, zb, acc, c, s):
    for off, sz in _ROWCHUNKS:
        r0 = s * RPT + off
        pltpu.sync_copy(acc.at[pl.ds(r0, sz), :], zb.at[pl.ds(0, sz), :])
        pltpu.sync_copy(zb.at[pl.ds(0, sz), :], out_hbm.at[c, pl.ds(r0, sz), :])


def _deg_body(edg_hbm, ones_hbm, zeros_hbm, out_hbm,
              i0, i1, i2, ones_v, dacc, s0, s1, s2):
    c = lax.axis_index("c")
    s = lax.axis_index("s")
    wid = s * NC + c
    cpw = edg_hbm.shape[0] // NW
    idx = [i0, i1, i2]
    sems = [s0, s1, s2]
    _zero_acc(zeros_hbm, ones_v, dacc, s)
    pltpu.sync_copy(ones_hbm, ones_v)
    plsc.subcore_barrier()

    base = wid * cpw

    def scat_start(b):
        pltpu.async_copy(ones_v, dacc.at[idx[b].at[1]], sems[b], add=True)

    def scat_wait(b):
        pltpu.make_async_copy(ones_v, dacc.at[idx[b].at[1]], sems[b]).wait()

    def step(t, carry):
        for b in range(NBUF):
            g = t * NBUF + b

            @pl.when(t > 0)
            def _():
                scat_wait(b)

            pltpu.sync_copy(edg_hbm.at[base + g], idx[b])
            scat_start(b)
        return carry

    lax.fori_loop(0, cpw // NBUF, step, 0)
    for b in range(NBUF):
        scat_wait(b)
    plsc.subcore_barrier()
    _copy_out(out_hbm, ones_v, dacc, c, s)


_deg_call = functools.partial(
    pl.kernel,
    out_type=jax.ShapeDtypeStruct((NC, NPAD, D), jnp.float32),
    mesh=_mesh,
    scratch_types=[
        pltpu.VMEM((2, CH), jnp.int32),
        pltpu.VMEM((2, CH), jnp.int32),
        pltpu.VMEM((2, CH), jnp.int32),
        pltpu.VMEM((CH, D), jnp.float32),
        pltpu.VMEM_SHARED((NPAD, D), jnp.float32),
        pltpu.SemaphoreType.DMA,
        pltpu.SemaphoreType.DMA,
        pltpu.SemaphoreType.DMA,
    ],
)(_deg_body)


def _make_agg_body(A, B):
    def _agg_body(u_hbm, edg_hbm, zeros_hbm, out_hbm,
                  i0, i1, i2, r0_, r1_, r2_, acc,
                  g0, g1, g2, s0, s1, s2):
        c = lax.axis_index("c")
        s = lax.axis_index("s")
        idx = [i0, i1, i2]
        rows = [r0_, r1_, r2_]
        semg = [g0, g1, g2]
        sems = [s0, s1, s2]
        _zero_acc(zeros_hbm, rows[0], acc, s)
        plsc.subcore_barrier()

        cnt = jnp.where(c == 0, A, B)
        base = jnp.where(c == 0, s * A, NS * A + s * B)

        def gath_start(b):
            pltpu.async_copy(u_hbm.at[idx[b].at[0]], rows[b], semg[b])

        def gath_wait(b):
            pltpu.make_async_copy(
                u_hbm.at[idx[b].at[0]], rows[b], semg[b]).wait()

        def scat_start(b):
            pltpu.async_copy(rows[b], acc.at[idx[b].at[1]], sems[b], add=True)

        def scat_wait(b):
            pltpu.make_async_copy(
                rows[b], acc.at[idx[b].at[1]], sems[b]).wait()

        @pl.when(cnt > 0)
        def _():
            for g in range(2):
                pltpu.sync_copy(edg_hbm.at[base + g], idx[g])
                gath_start(g)

        def step(t, carry):
            for b in range(NBUF):
                g = t * NBUF + b
                ga = g + 2
                bf = (b + 2) % NBUF

                @pl.when((ga < cnt) & (g >= 1))
                def _():
                    scat_wait(bf)

                @pl.when(ga < cnt)
                def _():
                    pltpu.sync_copy(edg_hbm.at[base + ga], idx[bf])
                    gath_start(bf)

                gath_wait(b)
                scat_start(b)
            return carry

        lax.fori_loop(0, cnt // NBUF, step, 0, unroll=False)

        @pl.when(cnt > 0)
        def _():
            for b in range(NBUF):
                scat_wait(b)

        plsc.subcore_barrier()
        _copy_out(out_hbm, rows[0], acc, c, s)

    return _agg_body


@functools.lru_cache(maxsize=None)
def _agg_call_for(A, B):
    return pl.kernel(
        _make_agg_body(A, B),
        out_type=jax.ShapeDtypeStruct((NC, NPAD, D), jnp.float32),
        mesh=_mesh,
        scratch_types=[
            pltpu.VMEM((2, CH), jnp.int32),
            pltpu.VMEM((2, CH), jnp.int32),
            pltpu.VMEM((2, CH), jnp.int32),
            pltpu.VMEM((CH, D), jnp.float32),
            pltpu.VMEM((CH, D), jnp.float32),
            pltpu.VMEM((CH, D), jnp.float32),
            pltpu.VMEM_SHARED((NPAD, D), jnp.float32),
            pltpu.SemaphoreType.DMA,
            pltpu.SemaphoreType.DMA,
            pltpu.SemaphoreType.DMA,
            pltpu.SemaphoreType.DMA,
            pltpu.SemaphoreType.DMA,
            pltpu.SemaphoreType.DMA,
        ],
    )


def _dinv_from(degp_ref):
    deg = degp_ref[0][:, 0:1] + degp_ref[1][:, 0:1] + 1.0
    return lax.rsqrt(deg)


def _tc_first(xp_ref, w_ref, degp_ref, u_ref):
    dinv = _dinv_from(degp_ref)
    h = jnp.dot(xp_ref[...], w_ref[...], preferred_element_type=jnp.float32)
    u_ref[...] = h * dinv


def _pairnorm(h, mask):
    h = h * mask
    cm = jnp.sum(h, axis=0, keepdims=True) * (1.0 / N)
    xc = (h - cm) * mask
    r2 = jnp.sum(xc * xc) * (1.0 / N)
    return xc * lax.rsqrt(1e-6 + r2)


def _tc_mid(p_ref, u_ref, degp_ref, b_ref, w_ref, mask_ref, out_ref):
    dinv = _dinv_from(degp_ref)
    h = (p_ref[0] + p_ref[1] + u_ref[...]) * dinv + b_ref[...]
    y = jnp.maximum(_pairnorm(h, mask_ref[...]), 0.0)
    out_ref[...] = jnp.dot(
        y, w_ref[...], preferred_element_type=jnp.float32) * dinv


def _tc_last(p_ref, u_ref, degp_ref, b_ref, mask_ref, out_ref):
    dinv = _dinv_from(degp_ref)
    h = (p_ref[0] + p_ref[1] + u_ref[...]) * dinv + b_ref[...]
    out_ref[...] = _pairnorm(h, mask_ref[...])


_of32 = jax.ShapeDtypeStruct((NPAD, D), jnp.float32)
_tc_first_call = pl.pallas_call(_tc_first, out_shape=_of32)
_tc_mid_call = pl.pallas_call(_tc_mid, out_shape=_of32)
_tc_last_call = pl.pallas_call(_tc_last, out_shape=_of32)


@jax.jit
def _run(x, edge_index, W0, b0, W1, b1, W2, b2):
    E = edge_index.shape[1]
    cpw = -(-E // (NW * CH))
    cpw = -(-cpw // NBUF) * NBUF
    epad = NW * cpw * CH
    pad_idx = N + (jnp.arange(epad - E, dtype=jnp.int32) % (NPAD - N))
    fill = jnp.stack([pad_idx, pad_idx])
    edg = jnp.concatenate([edge_index.astype(jnp.int32), fill], axis=1)
    edg = jnp.transpose(edg.reshape(2, NW * cpw, CH), (1, 0, 2))

    xp = jnp.pad(x, ((0, NPAD - N), (0, 0)))
    W2p = jnp.pad(W2, ((0, 0), (0, D - W2.shape[1])))
    b0p = jnp.reshape(b0, (1, D))
    b1p = jnp.reshape(b1, (1, D))
    b2p = jnp.reshape(jnp.pad(b2, (0, D - b2.shape[0])), (1, D))
    mask = (jnp.arange(NPAD) < N).astype(jnp.float32)[:, None]
    zeros = jnp.zeros((CH, D), jnp.float32)
    ones = jnp.ones((CH, D), jnp.float32)

    per_pair = 2 * cpw
    A = int(round(SPLIT0 * per_pair / NBUF)) * NBUF
    A = min(max(A, 0), per_pair)
    B = per_pair - A
    _agg = _agg_call_for(A, B)

    degp = _deg_call(edg, ones, zeros)

    u = _tc_first_call(xp, W0, degp)
    p = _agg(u, edg, zeros)
    u = _tc_mid_call(p, u, degp, b0p, W1, mask)
    p = _agg(u, edg, zeros)
    u = _tc_mid_call(p, u, degp, b1p, W2p, mask)
    p = _agg(u, edg, zeros)
    out = _tc_last_call(p, u, degp, b2p, mask)
    return out[:N, :121]


def kernel(x, edge_index, W0, b0, W1, b1, W2, b2):
    return _run(x, edge_index, W0, b0, W1, b1, W2, b2)


# speedup vs baseline: 4.2803x; 1.0175x over previous
import functools

import jax
import jax.numpy as jnp
from jax import lax
from jax.experimental import pallas as pl
from jax.experimental.pallas import tpu as pltpu
from jax.experimental.pallas import tpu_sc as plsc

N = 10000
D = 128
NC = 2
NS = 16
NW = NC * NS
CH = 128
NPAD = 10112
RPT = NPAD // NS
NBUF = 3
SPLIT0 = 0.5
_ROWCHUNKS = [(off, min(CH, RPT - off)) for off in range(0, RPT, CH)]

_mesh = plsc.VectorSubcoreMesh(core_axis_name="c", subcore_axis_name="s")


def _zero_acc(zeros_hbm, zb, acc, s, sems):
    pltpu.sync_copy(zeros_hbm, zb)
    nb = len(sems)

    def d(k):
        off, sz = _ROWCHUNKS[k]
        return (zb.at[pl.ds(0, sz), :],
                acc.at[pl.ds(s * RPT + off, sz), :], sems[k % nb])

    for k in range(len(_ROWCHUNKS)):
        pltpu.async_copy(*d(k))
    for k in range(len(_ROWCHUNKS)):
        pltpu.make_async_copy(*d(k)).wait()


def _copy_out(out_hbm, bufs, acc, c, s, sems):
    n = len(_ROWCHUNKS)
    nb = len(bufs)

    def d1(k):
        off, sz = _ROWCHUNKS[k]
        return (acc.at[pl.ds(s * RPT + off, sz), :],
                bufs[k % nb].at[pl.ds(0, sz), :], sems[k % nb])

    def d2(k):
        off, sz = _ROWCHUNKS[k]
        return (bufs[k % nb].at[pl.ds(0, sz), :],
                out_hbm.at[c, pl.ds(s * RPT + off, sz), :], sems[k % nb])

    for k in range(n):
        if k >= nb:
            pltpu.make_async_copy(*d2(k - nb)).wait()
        pltpu.async_copy(*d1(k))
        pltpu.make_async_copy(*d1(k)).wait()
        pltpu.async_copy(*d2(k))
    for k in range(max(0, n - nb), n):
        pltpu.make_async_copy(*d2(k)).wait()


def _deg_body(edg_hbm, ones_hbm, zeros_hbm, out_hbm,
              i0, i1, i2, ones_v, zb1, zb2, dacc, s0, s1, s2):
    c = lax.axis_index("c")
    s = lax.axis_index("s")
    wid = s * NC + c
    cpw = edg_hbm.shape[0] // NW
    idx = [i0, i1, i2]
    sems = [s0, s1, s2]
    _zero_acc(zeros_hbm, ones_v, dacc, s, sems)
    pltpu.sync_copy(ones_hbm, ones_v)
    plsc.subcore_barrier()

    base = wid * cpw

    def scat_start(b):
        pltpu.async_copy(ones_v, dacc.at[idx[b].at[1]], sems[b], add=True)

    def scat_wait(b):
        pltpu.make_async_copy(ones_v, dacc.at[idx[b].at[1]], sems[b]).wait()

    def step(t, carry):
        for b in range(NBUF):
            g = t * NBUF + b

            @pl.when(t > 0)
            def _():
                scat_wait(b)

            pltpu.sync_copy(edg_hbm.at[base + g], idx[b])
            scat_start(b)
        return carry

    lax.fori_loop(0, cpw // NBUF, step, 0)
    for b in range(NBUF):
        scat_wait(b)
    plsc.subcore_barrier()
    _copy_out(out_hbm, [ones_v, zb1, zb2], dacc, c, s, sems)


_deg_call = functools.partial(
    pl.kernel,
    out_type=jax.ShapeDtypeStruct((NC, NPAD, D), jnp.float32),
    mesh=_mesh,
    scratch_types=[
        pltpu.VMEM((2, CH), jnp.int32),
        pltpu.VMEM((2, CH), jnp.int32),
        pltpu.VMEM((2, CH), jnp.int32),
        pltpu.VMEM((CH, D), jnp.float32),
        pltpu.VMEM((CH, D), jnp.float32),
        pltpu.VMEM((CH, D), jnp.float32),
        pltpu.VMEM_SHARED((NPAD, D), jnp.float32),
        pltpu.SemaphoreType.DMA,
        pltpu.SemaphoreType.DMA,
        pltpu.SemaphoreType.DMA,
    ],
)(_deg_body)


def _make_agg_body(A, B):
    def _agg_body(u_hbm, edg_hbm, zeros_hbm, out_hbm,
                  i0, i1, i2, r0_, r1_, r2_, acc,
                  g0, g1, g2, s0, s1, s2):
        c = lax.axis_index("c")
        s = lax.axis_index("s")
        idx = [i0, i1, i2]
        rows = [r0_, r1_, r2_]
        semg = [g0, g1, g2]
        sems = [s0, s1, s2]
        _zero_acc(zeros_hbm, rows[0], acc, s, sems)
        plsc.subcore_barrier()

        cnt = jnp.where(c == 0, A, B)
        base = jnp.where(c == 0, s * A, NS * A + s * B)

        def gath_start(b):
            pltpu.async_copy(u_hbm.at[idx[b].at[0]], rows[b], semg[b])

        def gath_wait(b):
            pltpu.make_async_copy(
                u_hbm.at[idx[b].at[0]], rows[b], semg[b]).wait()

        def scat_start(b):
            pltpu.async_copy(rows[b], acc.at[idx[b].at[1]], sems[b], add=True)

        def scat_wait(b):
            pltpu.make_async_copy(
                rows[b], acc.at[idx[b].at[1]], sems[b]).wait()

        @pl.when(cnt > 0)
        def _():
            for g in range(2):
                pltpu.sync_copy(edg_hbm.at[base + g], idx[g])
                gath_start(g)

        def step(t, carry):
            for b in range(NBUF):
                g = t * NBUF + b
                ga = g + 2
                bf = (b + 2) % NBUF

                @pl.when((ga < cnt) & (g >= 1))
                def _():
                    scat_wait(bf)

                @pl.when(ga < cnt)
                def _():
                    pltpu.sync_copy(edg_hbm.at[base + ga], idx[bf])
                    gath_start(bf)

                gath_wait(b)
                scat_start(b)
            return carry

        lax.fori_loop(0, cnt // NBUF, step, 0, unroll=False)

        @pl.when(cnt > 0)
        def _():
            for b in range(NBUF):
                scat_wait(b)

        plsc.subcore_barrier()
        _copy_out(out_hbm, rows, acc, c, s, semg)

    return _agg_body


@functools.lru_cache(maxsize=None)
def _agg_call_for(A, B):
    return pl.kernel(
        _make_agg_body(A, B),
        out_type=jax.ShapeDtypeStruct((NC, NPAD, D), jnp.float32),
        mesh=_mesh,
        scratch_types=[
            pltpu.VMEM((2, CH), jnp.int32),
            pltpu.VMEM((2, CH), jnp.int32),
            pltpu.VMEM((2, CH), jnp.int32),
            pltpu.VMEM((CH, D), jnp.float32),
            pltpu.VMEM((CH, D), jnp.float32),
            pltpu.VMEM((CH, D), jnp.float32),
            pltpu.VMEM_SHARED((NPAD, D), jnp.float32),
            pltpu.SemaphoreType.DMA,
            pltpu.SemaphoreType.DMA,
            pltpu.SemaphoreType.DMA,
            pltpu.SemaphoreType.DMA,
            pltpu.SemaphoreType.DMA,
            pltpu.SemaphoreType.DMA,
        ],
    )


def _tc_first(xp_ref, w_ref, degp_ref, u_ref, dinv_ref):
    deg = degp_ref[0][:, 0:1] + degp_ref[1][:, 0:1] + 1.0
    dinv = lax.rsqrt(deg)
    dinv_ref[...] = dinv
    h = jnp.dot(xp_ref[...], w_ref[...], preferred_element_type=jnp.float32)
    u_ref[...] = h * dinv


def _pairnorm(h, mask):
    h = h * mask
    cm = jnp.sum(h, axis=0, keepdims=True) * (1.0 / N)
    xc = (h - cm) * mask
    r2 = jnp.sum(xc * xc) * (1.0 / N)
    return xc * lax.rsqrt(1e-6 + r2)


def _tc_mid(p_ref, u_ref, dinv_ref, b_ref, w_ref, mask_ref, out_ref):
    dinv = dinv_ref[...]
    h = (p_ref[0] + p_ref[1] + u_ref[...]) * dinv + b_ref[...]
    y = jnp.maximum(_pairnorm(h, mask_ref[...]), 0.0)
    out_ref[...] = jnp.dot(
        y, w_ref[...], preferred_element_type=jnp.float32) * dinv


def _tc_last(p_ref, u_ref, dinv_ref, b_ref, mask_ref, out_ref):
    dinv = dinv_ref[...]
    h = (p_ref[0] + p_ref[1] + u_ref[...]) * dinv + b_ref[...]
    out_ref[...] = _pairnorm(h, mask_ref[...])


_of32 = jax.ShapeDtypeStruct((NPAD, D), jnp.float32)
_dinvf32 = jax.ShapeDtypeStruct((NPAD, 1), jnp.float32)
_tc_first_call = pl.pallas_call(_tc_first, out_shape=(_of32, _dinvf32))
_tc_mid_call = pl.pallas_call(_tc_mid, out_shape=_of32)
_tc_last_call = pl.pallas_call(_tc_last, out_shape=_of32)


@jax.jit
def _run(x, edge_index, W0, b0, W1, b1, W2, b2):
    E = edge_index.shape[1]
    cpw = -(-E // (NW * CH))
    cpw = -(-cpw // NBUF) * NBUF
    epad = NW * cpw * CH
    pad_idx = N + (jnp.arange(epad - E, dtype=jnp.int32) % (NPAD - N))
    fill = jnp.stack([pad_idx, pad_idx])
    edg = jnp.concatenate([edge_index.astype(jnp.int32), fill], axis=1)
    edg = jnp.transpose(edg.reshape(2, NW * cpw, CH), (1, 0, 2))

    xp = jnp.pad(x, ((0, NPAD - N), (0, 0)))
    W2p = jnp.pad(W2, ((0, 0), (0, D - W2.shape[1])))
    b0p = jnp.reshape(b0, (1, D))
    b1p = jnp.reshape(b1, (1, D))
    b2p = jnp.reshape(jnp.pad(b2, (0, D - b2.shape[0])), (1, D))
    mask = (jnp.arange(NPAD) < N).astype(jnp.float32)[:, None]
    zeros = jnp.zeros((CH, D), jnp.float32)
    ones = jnp.ones((CH, D), jnp.float32)

    per_pair = 2 * cpw
    A = int(round(SPLIT0 * per_pair / NBUF)) * NBUF
    A = min(max(A, 0), per_pair)
    B = per_pair - A
    _agg = _agg_call_for(A, B)

    degp = _deg_call(edg, ones, zeros)

    u, dinv = _tc_first_call(xp, W0, degp)
    p = _agg(u, edg, zeros)
    u = _tc_mid_call(p, u, dinv, b0p, W1, mask)
    p = _agg(u, edg, zeros)
    u = _tc_mid_call(p, u, dinv, b1p, W2p, mask)
    p = _agg(u, edg, zeros)
    out = _tc_last_call(p, u, dinv, b2p, mask)
    return out[:N, :121]


def kernel(x, edge_index, W0, b0, W1, b1, W2, b2):
    return _run(x, edge_index, W0, b0, W1, b1, W2, b2)


# speedup vs baseline: 4.8747x; 1.1389x over previous
import functools

import jax
import jax.numpy as jnp
from jax import lax
from jax.experimental import pallas as pl
from jax.experimental.pallas import tpu as pltpu
from jax.experimental.pallas import tpu_sc as plsc

N = 10000
D = 128
NC = 2
NS = 16
NW = NC * NS
CH = 120
NPAD = 10112
RPT = NPAD // NS
NBUF = 3
SPLIT0 = 0.5
_ROWCHUNKS = [(off, min(CH, RPT - off)) for off in range(0, RPT, CH)]

_mesh = plsc.VectorSubcoreMesh(core_axis_name="c", subcore_axis_name="s")


def _zero_acc(zeros_hbm, zb, acc, s, sems):
    pltpu.sync_copy(zeros_hbm, zb)
    nb = len(sems)

    def d(k):
        off, sz = _ROWCHUNKS[k]
        return (zb.at[pl.ds(0, sz), :],
                acc.at[pl.ds(s * RPT + off, sz), :], sems[k % nb])

    for k in range(len(_ROWCHUNKS)):
        pltpu.async_copy(*d(k))
    for k in range(len(_ROWCHUNKS)):
        pltpu.make_async_copy(*d(k)).wait()


def _copy_out(out_hbm, bufs, acc, c, s, sems):
    n = len(_ROWCHUNKS)
    nb = len(bufs)

    def d1(k):
        off, sz = _ROWCHUNKS[k]
        return (acc.at[pl.ds(s * RPT + off, sz), :],
                bufs[k % nb].at[pl.ds(0, sz), :], sems[k % nb])

    def d2(k):
        off, sz = _ROWCHUNKS[k]
        return (bufs[k % nb].at[pl.ds(0, sz), :],
                out_hbm.at[c, pl.ds(s * RPT + off, sz), :], sems[k % nb])

    for k in range(n):
        if k >= nb:
            pltpu.make_async_copy(*d2(k - nb)).wait()
        pltpu.async_copy(*d1(k))
        pltpu.make_async_copy(*d1(k)).wait()
        pltpu.async_copy(*d2(k))
    for k in range(max(0, n - nb), n):
        pltpu.make_async_copy(*d2(k)).wait()


def _deg_body(edg_hbm, ones_hbm, zeros_hbm, out_hbm,
              i0, i1, i2, i3, i4, i5, ones_v, zb1, zb2, dacc,
              s0, s1, s2, y0, y1, y2):
    c = lax.axis_index("c")
    s = lax.axis_index("s")
    wid = s * NC + c
    cpw = edg_hbm.shape[0] // NW
    idx = [i0, i1, i2, i3, i4, i5]
    sems = [s0, s1, s2]
    semi = [y0, y1, y2]
    _zero_acc(zeros_hbm, ones_v, dacc, s, sems)
    pltpu.sync_copy(ones_hbm, ones_v)
    plsc.subcore_barrier()

    base = wid * cpw

    def idx_start(g, j):
        pltpu.async_copy(edg_hbm.at[base + g], idx[j], semi[j % NBUF])

    def idx_wait(g, j):
        pltpu.make_async_copy(
            edg_hbm.at[base + g], idx[j], semi[j % NBUF]).wait()

    def scat_start(j):
        pltpu.async_copy(ones_v, dacc.at[idx[j].at[1]],
                         sems[j % NBUF], add=True)

    def scat_wait(j):
        pltpu.make_async_copy(
            ones_v, dacc.at[idx[j].at[1]], sems[j % NBUF]).wait()

    for g in range(2):
        pltpu.sync_copy(edg_hbm.at[base + g], idx[g])

    def step(t, carry):
        for j in range(2 * NBUF):
            g = t * (2 * NBUF) + j

            @pl.when(g + 2 < cpw)
            def _():
                idx_start(g + 2, (j + 2) % 6)

            @pl.when(g >= NBUF)
            def _():
                scat_wait((j + NBUF) % 6)

            @pl.when(g >= 2)
            def _():
                idx_wait(g, j)

            scat_start(j)
        return carry

    lax.fori_loop(0, cpw // (2 * NBUF), step, 0)
    for j in range(NBUF):
        scat_wait(NBUF + j)
    plsc.subcore_barrier()
    _copy_out(out_hbm, [ones_v, zb1, zb2], dacc, c, s, sems)


_deg_call = functools.partial(
    pl.kernel,
    out_type=jax.ShapeDtypeStruct((NC, NPAD, D), jnp.float32),
    mesh=_mesh,
    scratch_types=[
        pltpu.VMEM((2, CH), jnp.int32),
        pltpu.VMEM((2, CH), jnp.int32),
        pltpu.VMEM((2, CH), jnp.int32),
        pltpu.VMEM((2, CH), jnp.int32),
        pltpu.VMEM((2, CH), jnp.int32),
        pltpu.VMEM((2, CH), jnp.int32),
        pltpu.VMEM((CH, D), jnp.float32),
        pltpu.VMEM((CH, D), jnp.float32),
        pltpu.VMEM((CH, D), jnp.float32),
        pltpu.VMEM_SHARED((NPAD, D), jnp.float32),
        pltpu.SemaphoreType.DMA,
        pltpu.SemaphoreType.DMA,
        pltpu.SemaphoreType.DMA,
        pltpu.SemaphoreType.DMA,
        pltpu.SemaphoreType.DMA,
        pltpu.SemaphoreType.DMA,
    ],
)(_deg_body)


def _make_agg_body(A, B):
    def _agg_body(u_hbm, edg_hbm, zeros_hbm, out_hbm,
                  i0, i1, i2, i3, i4, i5, r0_, r1_, r2_, acc,
                  g0, g1, g2, s0, s1, s2, y0, y1, y2):
        c = lax.axis_index("c")
        s = lax.axis_index("s")
        idx = [i0, i1, i2, i3, i4, i5]
        rows = [r0_, r1_, r2_]
        semg = [g0, g1, g2]
        sems = [s0, s1, s2]
        semi = [y0, y1, y2]
        _zero_acc(zeros_hbm, rows[0], acc, s, sems)
        plsc.subcore_barrier()

        cnt = jnp.where(c == 0, A, B)
        base = jnp.where(c == 0, s * A, NS * A + s * B)

        def idx_start(g, j):
            pltpu.async_copy(edg_hbm.at[base + g], idx[j], semi[j % NBUF])

        def idx_wait(g, j):
            pltpu.make_async_copy(
                edg_hbm.at[base + g], idx[j], semi[j % NBUF]).wait()

        def gath_start(j):
            pltpu.async_copy(u_hbm.at[idx[j].at[0]],
                             rows[j % NBUF], semg[j % NBUF])

        def gath_wait(j):
            pltpu.make_async_copy(u_hbm.at[idx[j].at[0]],
                                  rows[j % NBUF], semg[j % NBUF]).wait()

        def scat_start(j):
            pltpu.async_copy(rows[j % NBUF], acc.at[idx[j].at[1]],
                             sems[j % NBUF], add=True)

        def scat_wait(j):
            pltpu.make_async_copy(rows[j % NBUF], acc.at[idx[j].at[1]],
                                  sems[j % NBUF]).wait()

        @pl.when(cnt > 0)
        def _():
            for g in range(3):
                pltpu.sync_copy(edg_hbm.at[base + g], idx[g])
            for g in range(2):
                gath_start(g)

        def step(t, carry):
            for j in range(2 * NBUF):
                g = t * (2 * NBUF) + j
                ga = g + 2

                @pl.when(g + 3 < cnt)
                def _():
                    idx_start(g + 3, (j + 3) % 6)

                @pl.when((ga < cnt) & (g >= 1))
                def _():
                    scat_wait((j + 5) % 6)

                @pl.when((ga < cnt) & (g >= 1))
                def _():
                    idx_wait(ga, (j + 2) % 6)

                @pl.when(ga < cnt)
                def _():
                    gath_start((j + 2) % 6)

                gath_wait(j)
                scat_start(j)
            return carry

        lax.fori_loop(0, cnt // (2 * NBUF), step, 0, unroll=False)

        @pl.when(cnt > 0)
        def _():
            for j in range(NBUF):
                scat_wait(NBUF + j)

        plsc.subcore_barrier()
        _copy_out(out_hbm, rows, acc, c, s, semg)

    return _agg_body


@functools.lru_cache(maxsize=None)
def _agg_call_for(A, B):
    return pl.kernel(
        _make_agg_body(A, B),
        out_type=jax.ShapeDtypeStruct((NC, NPAD, D), jnp.float32),
        mesh=_mesh,
        scratch_types=[
            pltpu.VMEM((2, CH), jnp.int32),
            pltpu.VMEM((2, CH), jnp.int32),
            pltpu.VMEM((2, CH), jnp.int32),
            pltpu.VMEM((2, CH), jnp.int32),
            pltpu.VMEM((2, CH), jnp.int32),
            pltpu.VMEM((2, CH), jnp.int32),
            pltpu.VMEM((CH, D), jnp.float32),
            pltpu.VMEM((CH, D), jnp.float32),
            pltpu.VMEM((CH, D), jnp.float32),
            pltpu.VMEM_SHARED((NPAD, D), jnp.float32),
            pltpu.SemaphoreType.DMA,
            pltpu.SemaphoreType.DMA,
            pltpu.SemaphoreType.DMA,
            pltpu.SemaphoreType.DMA,
            pltpu.SemaphoreType.DMA,
            pltpu.SemaphoreType.DMA,
            pltpu.SemaphoreType.DMA,
            pltpu.SemaphoreType.DMA,
            pltpu.SemaphoreType.DMA,
        ],
    )


def _tc_first(xp_ref, w_ref, degp_ref, u_ref, dinv_ref):
    deg = degp_ref[0][:, 0:1] + degp_ref[1][:, 0:1] + 1.0
    dinv = lax.rsqrt(deg)
    dinv_ref[...] = dinv
    h = jnp.dot(xp_ref[...], w_ref[...], preferred_element_type=jnp.float32)
    u_ref[...] = h * dinv


def _pairnorm(h, mask):
    h = h * mask
    cm = jnp.sum(h, axis=0, keepdims=True) * (1.0 / N)
    xc = (h - cm) * mask
    r2 = jnp.sum(xc * xc) * (1.0 / N)
    return xc * lax.rsqrt(1e-6 + r2)


def _tc_mid(p_ref, u_ref, dinv_ref, b_ref, w_ref, mask_ref, out_ref):
    dinv = dinv_ref[...]
    h = (p_ref[0] + p_ref[1] + u_ref[...]) * dinv + b_ref[...]
    y = jnp.maximum(_pairnorm(h, mask_ref[...]), 0.0)
    out_ref[...] = jnp.dot(
        y, w_ref[...], preferred_element_type=jnp.float32) * dinv


def _tc_last(p_ref, u_ref, dinv_ref, b_ref, mask_ref, out_ref):
    dinv = dinv_ref[...]
    h = (p_ref[0] + p_ref[1] + u_ref[...]) * dinv + b_ref[...]
    out_ref[...] = _pairnorm(h, mask_ref[...])


_of32 = jax.ShapeDtypeStruct((NPAD, D), jnp.float32)
_dinvf32 = jax.ShapeDtypeStruct((NPAD, 1), jnp.float32)
_tc_first_call = pl.pallas_call(_tc_first, out_shape=(_of32, _dinvf32))
_tc_mid_call = pl.pallas_call(_tc_mid, out_shape=_of32)
_tc_last_call = pl.pallas_call(_tc_last, out_shape=_of32)


@jax.jit
def _run(x, edge_index, W0, b0, W1, b1, W2, b2):
    E = edge_index.shape[1]
    cpw = -(-E // (NW * CH))
    cpw = -(-cpw // (2 * NBUF)) * (2 * NBUF)
    epad = NW * cpw * CH
    pad_idx = N + (jnp.arange(epad - E, dtype=jnp.int32) % (NPAD - N))
    fill = jnp.stack([pad_idx, pad_idx])
    edg = jnp.concatenate([edge_index.astype(jnp.int32), fill], axis=1)
    edg = jnp.transpose(edg.reshape(2, NW * cpw, CH), (1, 0, 2))

    xp = jnp.pad(x, ((0, NPAD - N), (0, 0)))
    W2p = jnp.pad(W2, ((0, 0), (0, D - W2.shape[1])))
    b0p = jnp.reshape(b0, (1, D))
    b1p = jnp.reshape(b1, (1, D))
    b2p = jnp.reshape(jnp.pad(b2, (0, D - b2.shape[0])), (1, D))
    mask = (jnp.arange(NPAD) < N).astype(jnp.float32)[:, None]
    zeros = jnp.zeros((CH, D), jnp.float32)
    ones = jnp.ones((CH, D), jnp.float32)

    per_pair = 2 * cpw
    A = int(round(SPLIT0 * per_pair / (2 * NBUF))) * (2 * NBUF)
    A = min(max(A, 0), per_pair)
    B = per_pair - A
    _agg = _agg_call_for(A, B)

    degp = _deg_call(edg, ones, zeros)

    u, dinv = _tc_first_call(xp, W0, degp)
    p = _agg(u, edg, zeros)
    u = _tc_mid_call(p, u, dinv, b0p, W1, mask)
    p = _agg(u, edg, zeros)
    u = _tc_mid_call(p, u, dinv, b1p, W2p, mask)
    p = _agg(u, edg, zeros)
    out = _tc_last_call(p, u, dinv, b2p, mask)
    return out[:N, :121]


def kernel(x, edge_index, W0, b0, W1, b1, W2, b2):
    return _run(x, edge_index, W0, b0, W1, b1, W2, b2)
